# baseline (speedup 1.0000x reference)
import functools

import jax
import jax.numpy as jnp
from jax import lax
from jax.experimental import pallas as pl
from jax.experimental.pallas import tpu as pltpu
from jax.experimental.pallas import tpu_sc as plsc

_NC = 2
_NS = 16
_NW = _NC * _NS
_SB = 128
_ET = 10240
_L = 16
_NP = 10240


def _sc_mesh():
    return plsc.VectorSubcoreMesh(core_axis_name="c", subcore_axis_name="s")


def _sc_params():
    return pltpu.CompilerParams(use_tc_tiling_on_sc=False)


def _deg_body(nb, dst3, out, dst2d, ones_v, stage, acc):
    c = lax.axis_index("c")
    s = lax.axis_index("s")
    w = c * _NS + s
    stripe = _NP // _NS
    nchunk = stripe // 128

    zeros = jnp.zeros((_L,), jnp.float32)
    ones = jnp.ones((_L,), jnp.float32)
    for i in range(128):
        stage[i, :] = zeros
    for i in range(_SB):
        ones_v[i, :] = ones
    for k in range(nchunk):
        pltpu.sync_copy(stage, acc.at[pl.ds(s * stripe + k * 128, 128)])
    plsc.subcore_barrier()

    pltpu.sync_copy(dst3.at[w], dst2d)

    def body(j, carry):
        pltpu.sync_copy(ones_v, acc.at[dst2d.at[j]], add=True)
        return carry

    lax.fori_loop(0, nb, body, 0)
    plsc.subcore_barrier()

    for k in range(nchunk):
        r0 = s * stripe + k * 128
        pltpu.sync_copy(acc.at[pl.ds(r0, 128)], stage)
        pltpu.sync_copy(stage, out.at[c, pl.ds(r0, 128)])


def _sc_degree(dst3, nb):
    body = functools.partial(_deg_body, nb)
    k = pl.kernel(
        body,
        out_type=jax.ShapeDtypeStruct((_NC, _NP, _L), jnp.float32),
        mesh=_sc_mesh(),
        compiler_params=_sc_params(),
        scratch_types=[
            pltpu.VMEM((nb, _SB), jnp.int32),
            pltpu.VMEM((_SB, _L), jnp.float32),
            pltpu.VMEM((128, _L), jnp.float32),
            pltpu.VMEM_SHARED((_NP, _L), jnp.float32),
        ],
    )
    return k(dst3)


def _agg_body(nb, table, src3, dst3, out, src2d, dst2d, rows0, rows1, stage,
              acc, sem):
    c = lax.axis_index("c")
    s = lax.axis_index("s")
    w = c * _NS + s
    stripe = _NP // _NS
    nchunk = stripe // 128

    zeros = jnp.zeros((_L,), jnp.float32)
    for i in range(128):
        for cc in range(4):
            stage[i, pl.ds(cc * _L, _L)] = zeros
    for k in range(nchunk):
        pltpu.sync_copy(stage, acc.at[pl.ds(s * stripe + k * 128, 128)])
    plsc.subcore_barrier()

    pltpu.sync_copy(src3.at[w], src2d)
    pltpu.sync_copy(dst3.at[w], dst2d)

    def body(j, carry):
        pltpu.async_copy(table.at[src2d.at[j]], rows0, sem).wait()
        pltpu.sync_copy(rows0, acc.at[dst2d.at[j]], add=True)
        return carry

    lax.fori_loop(0, nb, body, 0)
    plsc.subcore_barrier()

    for k in range(nchunk):
        r0 = s * stripe + k * 128
        pltpu.sync_copy(acc.at[pl.ds(r0, 128)], stage)
        pltpu.sync_copy(stage, out.at[c, pl.ds(r0, 128)])


def _sc_aggregate(table, src3, dst3, nb):
    body = functools.partial(_agg_body, nb)
    k = pl.kernel(
        body,
        out_type=jax.ShapeDtypeStruct((_NC, _NP, 64), jnp.float32),
        mesh=_sc_mesh(),
        compiler_params=_sc_params(),
        scratch_types=[
            pltpu.VMEM((nb, _SB), jnp.int32),
            pltpu.VMEM((nb, _SB), jnp.int32),
            pltpu.VMEM((_SB, 64), jnp.float32),
            pltpu.VMEM((_SB, 64), jnp.float32),
            pltpu.VMEM((128, 64), jnp.float32),
            pltpu.VMEM_SHARED((_NP, 64), jnp.float32),
            pltpu.SemaphoreType.DMA,
        ],
    )
    return k(table, src3, dst3)


def _mm_body(x_ref, w_ref, o_ref):
    o_ref[...] = jnp.dot(x_ref[...], w_ref[...],
                         preferred_element_type=jnp.float32)


def _scale_body(degp_ref, h1_ref, dis_ref, hs1_ref):
    n = h1_ref.shape[0]
    degp = degp_ref[...]
    deg = degp[0, :n, 0] + degp[1, :n, 0] + 1.0
    dis = lax.rsqrt(deg)
    dis_ref[...] = dis
    hs1_ref[...] = h1_ref[...] * dis[:, None]


def _hidden_body(rawp_ref, h1_ref, dis_ref, b1_ref, h_ref, hs2_ref):
    n = h1_ref.shape[0]
    rawp = rawp_ref[...]
    raw = rawp[0, :n] + rawp[1, :n]
    dis = dis_ref[...]
    a1 = dis[:, None] * raw + (dis * dis)[:, None] * h1_ref[...] \
        + b1_ref[...][None, :]
    h = jnp.maximum(a1, 0.0)
    h_ref[...] = h
    hs2_ref[...] = h * dis[:, None]


def _head_body(rawp_ref, h_ref, dis_ref, wmu_ref, bmu_ref, wvar_ref,
               bvar_ref, eps_ref, zm_ref, zv_ref, z_ref):
    n = h_ref.shape[0]
    rawp = rawp_ref[...]
    raw = rawp[0, :n] + rawp[1, :n]
    dis = dis_ref[...]
    a2 = dis[:, None] * raw + (dis * dis)[:, None] * h_ref[...]
    zm = jnp.dot(a2, wmu_ref[...], preferred_element_type=jnp.float32) \
        + bmu_ref[...][None, :]
    pv = jnp.dot(a2, wvar_ref[...], preferred_element_type=jnp.float32) \
        + bvar_ref[...][None, :]
    zv = jnp.maximum(pv, 0.0) + jnp.log(1.0 + jnp.exp(-jnp.abs(pv)))
    zm_ref[...] = zm
    zv_ref[...] = zv
    z_ref[...] = zm + zv * eps_ref[...]


def kernel(x, edge_index, W1, b1, Wmu, bmu, Wvar, bvar):
    n, d = x.shape
    e = edge_index.shape[1]
    h = W1.shape[1]
    z = Wmu.shape[1]
    nb = _ET // _SB
    pad = _NW * _ET - e
    srcp = jnp.concatenate([edge_index[0],
                            jnp.zeros((pad,), edge_index.dtype)])
    dstp = jnp.concatenate([edge_index[1],
                            jnp.full((pad,), n, edge_index.dtype)])
    src3 = srcp.reshape(_NW, nb, _SB)
    dst3 = dstp.reshape(_NW, nb, _SB)

    f32 = jnp.float32
    h1 = pl.pallas_call(
        _mm_body, out_shape=jax.ShapeDtypeStruct((n, h), f32))(x, W1)

    degp = _sc_degree(dst3, nb)

    dis, hs1 = pl.pallas_call(
        _scale_body,
        out_shape=(jax.ShapeDtypeStruct((n,), f32),
                   jax.ShapeDtypeStruct((n, h), f32)))(degp, h1)

    raw1p = _sc_aggregate(hs1, src3, dst3, nb)

    hh, hs2 = pl.pallas_call(
        _hidden_body,
        out_shape=(jax.ShapeDtypeStruct((n, h), f32),
                   jax.ShapeDtypeStruct((n, h), f32)))(raw1p, h1, dis, b1)

    raw2p = _sc_aggregate(hs2, src3, dst3, nb)

    eps = jax.random.normal(jax.random.key(42), (n, z), f32)
    zm, zv, zz = pl.pallas_call(
        _head_body,
        out_shape=(jax.ShapeDtypeStruct((n, z), f32),
                   jax.ShapeDtypeStruct((n, z), f32),
                   jax.ShapeDtypeStruct((n, z), f32)))(
        raw2p, hh, dis, Wmu, bmu, Wvar, bvar, eps)
    return (zm, zv, zz)


# speedup vs baseline: 1.0195x; 1.0195x over previous
import functools

import jax
import jax.numpy as jnp
from jax import lax
from jax.experimental import pallas as pl
from jax.experimental.pallas import tpu as pltpu
from jax.experimental.pallas import tpu_sc as plsc

_NC = 2
_NS = 16
_NW = _NC * _NS
_SB = 80
_ET = 10240
_L = 16
_NP = 10240


def _sc_mesh():
    return plsc.VectorSubcoreMesh(core_axis_name="c", subcore_axis_name="s")


def _sc_params():
    return pltpu.CompilerParams(use_tc_tiling_on_sc=False)


def _deg_body(nb, dst3, out, dst2d, ones_v, stage, acc):
    c = lax.axis_index("c")
    s = lax.axis_index("s")
    w = c * _NS + s
    stripe = _NP // _NS
    nchunk = stripe // 128

    zeros = jnp.zeros((_L,), jnp.float32)
    ones = jnp.ones((_L,), jnp.float32)
    for i in range(128):
        stage[i, :] = zeros
    for i in range(_SB):
        ones_v[i, :] = ones
    for k in range(nchunk):
        pltpu.sync_copy(stage, acc.at[pl.ds(s * stripe + k * 128, 128)])
    plsc.subcore_barrier()

    pltpu.sync_copy(dst3.at[w], dst2d)

    def body(j, carry):
        pltpu.sync_copy(ones_v, acc.at[dst2d.at[j]], add=True)
        return carry

    lax.fori_loop(0, nb, body, 0)
    plsc.subcore_barrier()

    for k in range(nchunk):
        r0 = s * stripe + k * 128
        pltpu.sync_copy(acc.at[pl.ds(r0, 128)], stage)
        pltpu.sync_copy(stage, out.at[c, pl.ds(r0, 128)])


def _sc_degree(dst3, nb):
    body = functools.partial(_deg_body, nb)
    k = pl.kernel(
        body,
        out_type=jax.ShapeDtypeStruct((_NC, _NP, _L), jnp.float32),
        mesh=_sc_mesh(),
        compiler_params=_sc_params(),
        scratch_types=[
            pltpu.VMEM((nb, _SB), jnp.int32),
            pltpu.VMEM((_SB, _L), jnp.float32),
            pltpu.VMEM((128, _L), jnp.float32),
            pltpu.VMEM_SHARED((_NP, _L), jnp.float32),
        ],
    )
    return k(dst3)


def _agg_body(nb, table, src3, dst3, out, src2d, dst2d, rows0, rows1, stage,
              acc, sem):
    c = lax.axis_index("c")
    s = lax.axis_index("s")
    w = c * _NS + s
    stripe = _NP // _NS
    nchunk = stripe // 128

    zeros = jnp.zeros((_L,), jnp.float32)
    for i in range(128):
        for cc in range(4):
            stage[i, pl.ds(cc * _L, _L)] = zeros
    for k in range(nchunk):
        pltpu.sync_copy(stage, acc.at[pl.ds(s * stripe + k * 128, 128)])
    plsc.subcore_barrier()

    pltpu.sync_copy(src3.at[w], src2d)
    pltpu.sync_copy(dst3.at[w], dst2d)

    pltpu.async_copy(table.at[src2d.at[0]], rows0, sem)

    def body(i, carry):
        j = 2 * i
        pltpu.make_async_copy(table.at[src2d.at[j]], rows0, sem).wait()
        pltpu.async_copy(table.at[src2d.at[j + 1]], rows1, sem)
        pltpu.sync_copy(rows0, acc.at[dst2d.at[j]], add=True)
        pltpu.make_async_copy(table.at[src2d.at[j]], rows1, sem).wait()

        @pl.when(j + 2 < nb)
        def _():
            pltpu.async_copy(table.at[src2d.at[j + 2]], rows0, sem)

        pltpu.sync_copy(rows1, acc.at[dst2d.at[j + 1]], add=True)
        return carry

    lax.fori_loop(0, nb // 2, body, 0)
    plsc.subcore_barrier()

    for k in range(nchunk):
        r0 = s * stripe + k * 128
        pltpu.sync_copy(acc.at[pl.ds(r0, 128)], stage)
        pltpu.sync_copy(stage, out.at[c, pl.ds(r0, 128)])


def _sc_aggregate(table, src3, dst3, nb):
    body = functools.partial(_agg_body, nb)
    k = pl.kernel(
        body,
        out_type=jax.ShapeDtypeStruct((_NC, _NP, 64), jnp.float32),
        mesh=_sc_mesh(),
        compiler_params=_sc_params(),
        scratch_types=[
            pltpu.VMEM((nb, _SB), jnp.int32),
            pltpu.VMEM((nb, _SB), jnp.int32),
            pltpu.VMEM((_SB, 64), jnp.float32),
            pltpu.VMEM((_SB, 64), jnp.float32),
            pltpu.VMEM((128, 64), jnp.float32),
            pltpu.VMEM_SHARED((_NP, 64), jnp.float32),
            pltpu.SemaphoreType.DMA,
        ],
    )
    return k(table, src3, dst3)


def _mm_body(x_ref, w_ref, o_ref):
    o_ref[...] = jnp.dot(x_ref[...], w_ref[...],
                         preferred_element_type=jnp.float32)


def _scale_body(degp_ref, h1_ref, dis_ref, hs1_ref):
    n = h1_ref.shape[0]
    degp = degp_ref[...]
    deg = degp[0, :n, 0] + degp[1, :n, 0] + 1.0
    dis = lax.rsqrt(deg)
    dis_ref[...] = dis
    hs1_ref[...] = h1_ref[...] * dis[:, None]


def _hidden_body(rawp_ref, h1_ref, dis_ref, b1_ref, h_ref, hs2_ref):
    n = h1_ref.shape[0]
    rawp = rawp_ref[...]
    raw = rawp[0, :n] + rawp[1, :n]
    dis = dis_ref[...]
    a1 = dis[:, None] * raw + (dis * dis)[:, None] * h1_ref[...] \
        + b1_ref[...][None, :]
    h = jnp.maximum(a1, 0.0)
    h_ref[...] = h
    hs2_ref[...] = h * dis[:, None]


def _head_body(rawp_ref, h_ref, dis_ref, wmu_ref, bmu_ref, wvar_ref,
               bvar_ref, eps_ref, zm_ref, zv_ref, z_ref):
    n = h_ref.shape[0]
    rawp = rawp_ref[...]
    raw = rawp[0, :n] + rawp[1, :n]
    dis = dis_ref[...]
    a2 = dis[:, None] * raw + (dis * dis)[:, None] * h_ref[...]
    zm = jnp.dot(a2, wmu_ref[...], preferred_element_type=jnp.float32) \
        + bmu_ref[...][None, :]
    pv = jnp.dot(a2, wvar_ref[...], preferred_element_type=jnp.float32) \
        + bvar_ref[...][None, :]
    zv = jnp.maximum(pv, 0.0) + jnp.log(1.0 + jnp.exp(-jnp.abs(pv)))
    zm_ref[...] = zm
    zv_ref[...] = zv
    z_ref[...] = zm + zv * eps_ref[...]


def kernel(x, edge_index, W1, b1, Wmu, bmu, Wvar, bvar):
    n, d = x.shape
    e = edge_index.shape[1]
    h = W1.shape[1]
    z = Wmu.shape[1]
    nb = _ET // _SB
    pad = _NW * _ET - e
    srcp = jnp.concatenate([edge_index[0],
                            jnp.zeros((pad,), edge_index.dtype)])
    dstp = jnp.concatenate([edge_index[1],
                            jnp.full((pad,), n, edge_index.dtype)])
    src3 = srcp.reshape(_NW, nb, _SB)
    dst3 = dstp.reshape(_NW, nb, _SB)

    f32 = jnp.float32
    h1 = pl.pallas_call(
        _mm_body, out_shape=jax.ShapeDtypeStruct((n, h), f32))(x, W1)

    degp = _sc_degree(dst3, nb)

    dis, hs1 = pl.pallas_call(
        _scale_body,
        out_shape=(jax.ShapeDtypeStruct((n,), f32),
                   jax.ShapeDtypeStruct((n, h), f32)))(degp, h1)

    raw1p = _sc_aggregate(hs1, src3, dst3, nb)

    hh, hs2 = pl.pallas_call(
        _hidden_body,
        out_shape=(jax.ShapeDtypeStruct((n, h), f32),
                   jax.ShapeDtypeStruct((n, h), f32)))(raw1p, h1, dis, b1)

    raw2p = _sc_aggregate(hs2, src3, dst3, nb)

    eps = jax.random.normal(jax.random.key(42), (n, z), f32)
    zm, zv, zz = pl.pallas_call(
        _head_body,
        out_shape=(jax.ShapeDtypeStruct((n, z), f32),
                   jax.ShapeDtypeStruct((n, z), f32),
                   jax.ShapeDtypeStruct((n, z), f32)))(
        raw2p, hh, dis, Wmu, bmu, Wvar, bvar, eps)
    return (zm, zv, zz)


# speedup vs baseline: 1.0195x; 1.0000x over previous
import functools

import jax
import jax.numpy as jnp
from jax import lax
from jax.experimental import pallas as pl
from jax.experimental.pallas import tpu as pltpu
from jax.experimental.pallas import tpu_sc as plsc

_NC = 2
_NS = 16
_NW = _NC * _NS
_SB = 80
_ET = 10240
_L = 16
_NP = 10240


def _sc_mesh():
    return plsc.VectorSubcoreMesh(core_axis_name="c", subcore_axis_name="s")


def _sc_params():
    return pltpu.CompilerParams(use_tc_tiling_on_sc=False)


def _deg_body(nb, dst3, out, dst2d, ones_v, stage, acc):
    c = lax.axis_index("c")
    s = lax.axis_index("s")
    w = c * _NS + s
    stripe = _NP // _NS
    nchunk = stripe // 128

    zeros = jnp.zeros((_L,), jnp.float32)
    ones = jnp.ones((_L,), jnp.float32)
    for i in range(128):
        stage[i, :] = zeros
    for i in range(_SB):
        ones_v[i, :] = ones
    for k in range(nchunk):
        pltpu.sync_copy(stage, acc.at[pl.ds(s * stripe + k * 128, 128)])
    plsc.subcore_barrier()

    pltpu.sync_copy(dst3.at[w], dst2d)

    def body(j, carry):
        pltpu.sync_copy(ones_v, acc.at[dst2d.at[j]], add=True)
        return carry

    lax.fori_loop(0, nb, body, 0)
    plsc.subcore_barrier()

    for k in range(nchunk):
        r0 = s * stripe + k * 128
        pltpu.sync_copy(acc.at[pl.ds(r0, 128)], stage)
        pltpu.sync_copy(stage, out.at[c, pl.ds(r0, 128)])


def _sc_degree(dst3, nb):
    body = functools.partial(_deg_body, nb)
    k = pl.kernel(
        body,
        out_type=jax.ShapeDtypeStruct((_NC, _NP, _L), jnp.float32),
        mesh=_sc_mesh(),
        compiler_params=_sc_params(),
        scratch_types=[
            pltpu.VMEM((nb, _SB), jnp.int32),
            pltpu.VMEM((_SB, _L), jnp.float32),
            pltpu.VMEM((128, _L), jnp.float32),
            pltpu.VMEM_SHARED((_NP, _L), jnp.float32),
        ],
    )
    return k(dst3)


def _agg_body(nb, table, src3, dst3, out, src2d, dst2d, rows0, rows1, stage,
              acc, sem):
    c = lax.axis_index("c")
    s = lax.axis_index("s")
    w = c * _NS + s
    stripe = _NP // _NS
    nchunk = stripe // 128

    zeros = jnp.zeros((_L,), jnp.float32)
    for i in range(128):
        for cc in range(4):
            stage[i, pl.ds(cc * _L, _L)] = zeros
    for k in range(nchunk):
        pltpu.sync_copy(stage, acc.at[pl.ds(s * stripe + k * 128, 128)])
    plsc.subcore_barrier()

    pltpu.sync_copy(src3.at[w], src2d)
    pltpu.sync_copy(dst3.at[w], dst2d)

    pltpu.async_copy(table.at[src2d.at[0]], rows0, sem)

    def body(i, carry):
        j = 2 * i
        pltpu.make_async_copy(table.at[src2d.at[j]], rows0, sem).wait()
        pltpu.async_copy(table.at[src2d.at[j + 1]], rows1, sem)
        pltpu.sync_copy(rows0, acc.at[dst2d.at[j]], add=True)
        pltpu.make_async_copy(table.at[src2d.at[j]], rows1, sem).wait()

        @pl.when(j + 2 < nb)
        def _():
            pltpu.async_copy(table.at[src2d.at[j + 2]], rows0, sem)

        pltpu.sync_copy(rows1, acc.at[dst2d.at[j + 1]], add=True)
        return carry

    lax.fori_loop(0, nb // 2, body, 0)
    plsc.subcore_barrier()

    for k in range(nchunk):
        r0 = s * stripe + k * 128
        pltpu.sync_copy(acc.at[pl.ds(r0, 128)], stage)
        pltpu.sync_copy(stage, out.at[c, pl.ds(r0, 128)])


def _sc_aggregate(table, src3, dst3, nb):
    body = functools.partial(_agg_body, nb)
    k = pl.kernel(
        body,
        out_type=jax.ShapeDtypeStruct((_NC, _NP, 64), jnp.float32),
        mesh=_sc_mesh(),
        compiler_params=_sc_params(),
        scratch_types=[
            pltpu.VMEM((nb, _SB), jnp.int32),
            pltpu.VMEM((nb, _SB), jnp.int32),
            pltpu.VMEM((_SB, 64), jnp.float32),
            pltpu.VMEM((_SB, 64), jnp.float32),
            pltpu.VMEM((128, 64), jnp.float32),
            pltpu.VMEM_SHARED((_NP, 64), jnp.float32),
            pltpu.SemaphoreType.DMA,
        ],
    )
    return k(table, src3, dst3)


def _mm_body(x_ref, w_ref, o_ref):
    o_ref[...] = jnp.dot(x_ref[...], w_ref[...],
                         preferred_element_type=jnp.float32)


def _scale_body(degp_ref, h1_ref, dis_ref, hs1_ref):
    n = h1_ref.shape[0]
    degp = degp_ref[...]
    deg = degp[0, :n, 0] + degp[1, :n, 0] + 1.0
    dis = lax.rsqrt(deg)
    dis_ref[...] = dis
    hs1_ref[...] = h1_ref[...] * dis[:, None]


def _hidden_body(rawp_ref, h1_ref, dis_ref, b1_ref, h_ref, hs2_ref):
    n = h1_ref.shape[0]
    rawp = rawp_ref[...]
    raw = rawp[0, :n] + rawp[1, :n]
    dis = dis_ref[...]
    a1 = dis[:, None] * raw + (dis * dis)[:, None] * h1_ref[...] \
        + b1_ref[...][None, :]
    h = jnp.maximum(a1, 0.0)
    h_ref[...] = h
    hs2_ref[...] = h * dis[:, None]


def _head_body(rawp_ref, h_ref, dis_ref, wmu_ref, bmu_ref, wvar_ref,
               bvar_ref, eps_ref, zm_ref, zv_ref, z_ref):
    n = h_ref.shape[0]
    rawp = rawp_ref[...]
    raw = rawp[0, :n] + rawp[1, :n]
    dis = dis_ref[...]
    a2 = dis[:, None] * raw + (dis * dis)[:, None] * h_ref[...]
    zm = jnp.dot(a2, wmu_ref[...], preferred_element_type=jnp.float32) \
        + bmu_ref[...][None, :]
    pv = jnp.dot(a2, wvar_ref[...], preferred_element_type=jnp.float32) \
        + bvar_ref[...][None, :]
    zv = jnp.maximum(pv, 0.0) + jnp.log(1.0 + jnp.exp(-jnp.abs(pv)))
    zm_ref[...] = zm
    zv_ref[...] = zv
    z_ref[...] = zm + zv * eps_ref[...]


def kernel(x, edge_index, W1, b1, Wmu, bmu, Wvar, bvar):
    n, d = x.shape
    e = edge_index.shape[1]
    h = W1.shape[1]
    z = Wmu.shape[1]
    nb = _ET // _SB
    pad = _NW * _ET - e
    srcp = jnp.concatenate([edge_index[0],
                            jnp.zeros((pad,), edge_index.dtype)])
    pad_dst = n + jax.lax.rem(
        jnp.arange(pad, dtype=edge_index.dtype),
        jnp.asarray(_NP - n, edge_index.dtype))
    dstp = jnp.concatenate([edge_index[1], pad_dst])
    src3 = srcp.reshape(_NW, nb, _SB)
    dst3 = dstp.reshape(_NW, nb, _SB)

    f32 = jnp.float32
    h1 = pl.pallas_call(
        _mm_body, out_shape=jax.ShapeDtypeStruct((n, h), f32))(x, W1)

    degp = _sc_degree(dst3, nb)

    dis, hs1 = pl.pallas_call(
        _scale_body,
        out_shape=(jax.ShapeDtypeStruct((n,), f32),
                   jax.ShapeDtypeStruct((n, h), f32)))(degp, h1)

    raw1p = _sc_aggregate(hs1, src3, dst3, nb)

    hh, hs2 = pl.pallas_call(
        _hidden_body,
        out_shape=(jax.ShapeDtypeStruct((n, h), f32),
                   jax.ShapeDtypeStruct((n, h), f32)))(raw1p, h1, dis, b1)

    raw2p = _sc_aggregate(hs2, src3, dst3, nb)

    eps = jax.random.normal(jax.random.key(42), (n, z), f32)
    zm, zv, zz = pl.pallas_call(
        _head_body,
        out_shape=(jax.ShapeDtypeStruct((n, z), f32),
                   jax.ShapeDtypeStruct((n, z), f32),
                   jax.ShapeDtypeStruct((n, z), f32)))(
        raw2p, hh, dis, Wmu, bmu, Wvar, bvar, eps)
    return (zm, zv, zz)


# speedup vs baseline: 1.8781x; 1.8422x over previous
import functools

import jax
import jax.numpy as jnp
from jax import lax
from jax.experimental import pallas as pl
from jax.experimental.pallas import tpu as pltpu
from jax.experimental.pallas import tpu_sc as plsc

_NC = 2
_NS = 16
_NW = _NC * _NS
_SB = 80
_L = 16
_NP = 10240


def _sc_mesh():
    return plsc.VectorSubcoreMesh(core_axis_name="c", subcore_axis_name="s")


def _sc_params():
    return pltpu.CompilerParams(use_tc_tiling_on_sc=False)


def _deg_body(nb, dst3, out, dst2d, ones_v, stage, acc):
    c = lax.axis_index("c")
    s = lax.axis_index("s")
    w = c * _NS + s
    stripe = _NP // _NS
    nchunk = stripe // 128

    zeros = jnp.zeros((_L,), jnp.float32)
    ones = jnp.ones((_L,), jnp.float32)
    for i in range(128):
        stage[i, :] = zeros
    for i in range(_SB):
        ones_v[i, :] = ones
    for k in range(nchunk):
        pltpu.sync_copy(stage, acc.at[pl.ds(s * stripe + k * 128, 128)])
    plsc.subcore_barrier()

    pltpu.sync_copy(dst3.at[w], dst2d)

    def body(j, carry):
        pltpu.sync_copy(ones_v, acc.at[dst2d.at[j]], add=True)
        return carry

    lax.fori_loop(0, nb, body, 0)
    plsc.subcore_barrier()

    for k in range(nchunk):
        r0 = s * stripe + k * 128
        pltpu.sync_copy(acc.at[pl.ds(r0, 128)], stage)
        pltpu.sync_copy(stage, out.at[c, pl.ds(r0, 128)])


def _sc_degree(dst3, nb):
    body = functools.partial(_deg_body, nb)
    k = pl.kernel(
        body,
        out_type=jax.ShapeDtypeStruct((_NC, _NP, _L), jnp.float32),
        mesh=_sc_mesh(),
        compiler_params=_sc_params(),
        scratch_types=[
            pltpu.VMEM((nb, _SB), jnp.int32),
            pltpu.VMEM((_SB, _L), jnp.float32),
            pltpu.VMEM((128, _L), jnp.float32),
            pltpu.VMEM_SHARED((_NP, _L), jnp.float32),
        ],
    )
    return k(dst3)


def _agg_body(nb, table, src3, dst3, out, src2d, dst2d, rows0, rows1, stage,
              acc, sem):
    c = lax.axis_index("c")
    s = lax.axis_index("s")
    w = c * _NS + s
    stripe = _NP // _NS
    nchunk = stripe // 128

    zeros = jnp.zeros((_L,), jnp.float32)
    for i in range(128):
        for cc in range(4):
            stage[i, pl.ds(cc * _L, _L)] = zeros
    for k in range(nchunk):
        pltpu.sync_copy(stage, acc.at[pl.ds(s * stripe + k * 128, 128)])
    plsc.subcore_barrier()

    pltpu.sync_copy(src3.at[w], src2d)
    pltpu.sync_copy(dst3.at[w], dst2d)

    pltpu.async_copy(table.at[src2d.at[0]], rows0, sem)

    def body(i, carry):
        j = 2 * i
        pltpu.make_async_copy(table.at[src2d.at[j]], rows0, sem).wait()
        pltpu.async_copy(table.at[src2d.at[j + 1]], rows1, sem)
        pltpu.sync_copy(rows0, acc.at[dst2d.at[j]], add=True)
        pltpu.make_async_copy(table.at[src2d.at[j]], rows1, sem).wait()

        @pl.when(j + 2 < nb)
        def _():
            pltpu.async_copy(table.at[src2d.at[j + 2]], rows0, sem)

        pltpu.sync_copy(rows1, acc.at[dst2d.at[j + 1]], add=True)
        return carry

    lax.fori_loop(0, nb // 2, body, 0)
    pltpu.make_async_copy(table.at[src2d.at[0]], rows0, sem).wait()
    pltpu.sync_copy(rows0, acc.at[dst2d.at[nb - 1]], add=True)
    plsc.subcore_barrier()

    for k in range(nchunk):
        r0 = s * stripe + k * 128
        pltpu.sync_copy(acc.at[pl.ds(r0, 128)], stage)
        pltpu.sync_copy(stage, out.at[c, pl.ds(r0, 128)])


def _sc_aggregate(table, src3, dst3, nb):
    body = functools.partial(_agg_body, nb)
    k = pl.kernel(
        body,
        out_type=jax.ShapeDtypeStruct((_NC, _NP, 64), jnp.float32),
        mesh=_sc_mesh(),
        compiler_params=_sc_params(),
        scratch_types=[
            pltpu.VMEM((nb, _SB), jnp.int32),
            pltpu.VMEM((nb, _SB), jnp.int32),
            pltpu.VMEM((_SB, 64), jnp.float32),
            pltpu.VMEM((_SB, 64), jnp.float32),
            pltpu.VMEM((128, 64), jnp.float32),
            pltpu.VMEM_SHARED((_NP, 64), jnp.float32),
            pltpu.SemaphoreType.DMA,
        ],
    )
    return k(table, src3, dst3)


def _mm_body(x_ref, w_ref, o_ref):
    o_ref[...] = jnp.dot(x_ref[...], w_ref[...],
                         preferred_element_type=jnp.float32)


def _scale_body(degp_ref, h1_ref, dis_ref, hs1_ref):
    n = h1_ref.shape[0]
    degp = degp_ref[...]
    deg = degp[0, :n, 0] + degp[1, :n, 0] + 1.0
    dis = lax.rsqrt(deg)
    dis_ref[...] = dis
    hs1_ref[...] = h1_ref[...] * dis[:, None]


def _hidden_body(rawp_ref, h1_ref, dis_ref, b1_ref, h_ref, hs2_ref):
    n = h1_ref.shape[0]
    rawp = rawp_ref[...]
    raw = rawp[0, :n] + rawp[1, :n]
    dis = dis_ref[...]
    a1 = dis[:, None] * raw + (dis * dis)[:, None] * h1_ref[...] \
        + b1_ref[...][None, :]
    h = jnp.maximum(a1, 0.0)
    h_ref[...] = h
    hs2_ref[...] = h * dis[:, None]


def _head_body(rawp_ref, h_ref, dis_ref, wmu_ref, bmu_ref, wvar_ref,
               bvar_ref, eps_ref, zm_ref, zv_ref, z_ref):
    n = h_ref.shape[0]
    rawp = rawp_ref[...]
    raw = rawp[0, :n] + rawp[1, :n]
    dis = dis_ref[...]
    a2 = dis[:, None] * raw + (dis * dis)[:, None] * h_ref[...]
    zm = jnp.dot(a2, wmu_ref[...], preferred_element_type=jnp.float32) \
        + bmu_ref[...][None, :]
    pv = jnp.dot(a2, wvar_ref[...], preferred_element_type=jnp.float32) \
        + bvar_ref[...][None, :]
    zv = jnp.maximum(pv, 0.0) + jnp.log(1.0 + jnp.exp(-jnp.abs(pv)))
    zm_ref[...] = zm
    zv_ref[...] = zv
    z_ref[...] = zm + zv * eps_ref[...]


def kernel(x, edge_index, W1, b1, Wmu, bmu, Wvar, bvar):
    n, d = x.shape
    e = edge_index.shape[1]
    h = W1.shape[1]
    z = Wmu.shape[1]
    nb = e // (_NW * _SB)
    src3 = edge_index[0].reshape(_NW, nb, _SB)
    dst3 = edge_index[1].reshape(_NW, nb, _SB)

    f32 = jnp.float32
    h1 = pl.pallas_call(
        _mm_body, out_shape=jax.ShapeDtypeStruct((n, h), f32))(x, W1)

    degp = _sc_degree(dst3, nb)

    dis, hs1 = pl.pallas_call(
        _scale_body,
        out_shape=(jax.ShapeDtypeStruct((n,), f32),
                   jax.ShapeDtypeStruct((n, h), f32)))(degp, h1)

    raw1p = _sc_aggregate(hs1, src3, dst3, nb)

    hh, hs2 = pl.pallas_call(
        _hidden_body,
        out_shape=(jax.ShapeDtypeStruct((n, h), f32),
                   jax.ShapeDtypeStruct((n, h), f32)))(raw1p, h1, dis, b1)

    raw2p = _sc_aggregate(hs2, src3, dst3, nb)

    eps = jax.random.normal(jax.random.key(42), (n, z), f32)
    zm, zv, zz = pl.pallas_call(
        _head_body,
        out_shape=(jax.ShapeDtypeStruct((n, z), f32),
                   jax.ShapeDtypeStruct((n, z), f32),
                   jax.ShapeDtypeStruct((n, z), f32)))(
        raw2p, hh, dis, Wmu, bmu, Wvar, bvar, eps)
    return (zm, zv, zz)


# speedup vs baseline: 2.7990x; 1.4903x over previous
import functools

import jax
import jax.numpy as jnp
from jax import lax
from jax.experimental import pallas as pl
from jax.experimental.pallas import tpu as pltpu
from jax.experimental.pallas import tpu_sc as plsc

_NC = 2
_NS = 16
_NW = _NC * _NS
_SB = 80
_L = 16
_NP = 10240


def _sc_mesh():
    return plsc.VectorSubcoreMesh(core_axis_name="c", subcore_axis_name="s")


def _sc_params():
    return pltpu.CompilerParams(use_tc_tiling_on_sc=False)


def _deg_body(nb, dst3, out, dst2d, ones_v, stage, acc):
    c = lax.axis_index("c")
    s = lax.axis_index("s")
    w = c * _NS + s
    stripe = _NP // _NS
    nchunk = stripe // 128

    zeros = jnp.zeros((_L,), jnp.float32)
    ones = jnp.ones((_L,), jnp.float32)
    for i in range(128):
        stage[i, :] = zeros
    for i in range(_SB):
        ones_v[i, :] = ones
    for k in range(nchunk):
        pltpu.sync_copy(stage, acc.at[pl.ds(s * stripe + k * 128, 128)])
    plsc.subcore_barrier()

    pltpu.sync_copy(dst3.at[w], dst2d)

    def body(j, carry):
        pltpu.sync_copy(ones_v, acc.at[dst2d.at[j]], add=True)
        return carry

    lax.fori_loop(0, nb, body, 0)
    plsc.subcore_barrier()

    for k in range(nchunk):
        r0 = s * stripe + k * 128
        pltpu.sync_copy(acc.at[pl.ds(r0, 128)], stage)
        pltpu.sync_copy(stage, out.at[c, pl.ds(r0, 128)])


def _sc_degree(dst3, nb):
    body = functools.partial(_deg_body, nb)
    k = pl.kernel(
        body,
        out_type=jax.ShapeDtypeStruct((_NC, _NP, _L), jnp.float32),
        mesh=_sc_mesh(),
        compiler_params=_sc_params(),
        scratch_types=[
            pltpu.VMEM((nb, _SB), jnp.int32),
            pltpu.VMEM((_SB, _L), jnp.float32),
            pltpu.VMEM((128, _L), jnp.float32),
            pltpu.VMEM_SHARED((_NP, _L), jnp.float32),
        ],
    )
    return k(dst3)


def _agg_body(nb, table, src3, dst3, out, src2d, dst2d, rows0, rows1, rows2,
              rows3, stage, acc, sem):
    c = lax.axis_index("c")
    s = lax.axis_index("s")
    w = c * _NS + s
    stripe = _NP // _NS
    nchunk = stripe // 128
    bufs = [rows0, rows1, rows2, rows3]
    depth = 3

    zeros = jnp.zeros((_L,), jnp.float32)
    for i in range(128):
        for cc in range(4):
            stage[i, pl.ds(cc * _L, _L)] = zeros
    for k in range(nchunk):
        pltpu.sync_copy(stage, acc.at[pl.ds(s * stripe + k * 128, 128)])
    plsc.subcore_barrier()

    pltpu.sync_copy(src3.at[w], src2d)
    pltpu.sync_copy(dst3.at[w], dst2d)

    for j in range(depth):
        pltpu.async_copy(table.at[src2d.at[j]], bufs[j], sem)

    nb4 = nb // 4
    rem = nb - 4 * nb4

    def body(i, carry):
        j = 4 * i
        for k in range(4):
            b = bufs[k]
            pltpu.make_async_copy(table.at[src2d.at[j]], b, sem).wait()

            @pl.when(j + k + depth < nb)
            def _():
                pltpu.async_copy(
                    table.at[src2d.at[j + k + depth]], bufs[(k + depth) % 4],
                    sem)

            pltpu.sync_copy(b, acc.at[dst2d.at[j + k]], add=True)
        return carry

    lax.fori_loop(0, nb4, body, 0)
    for t in range(rem):
        j = 4 * nb4 + t
        b = bufs[j % 4]
        pltpu.make_async_copy(table.at[src2d.at[0]], b, sem).wait()
        pltpu.sync_copy(b, acc.at[dst2d.at[j]], add=True)
    plsc.subcore_barrier()

    for k in range(nchunk):
        r0 = s * stripe + k * 128
        pltpu.sync_copy(acc.at[pl.ds(r0, 128)], stage)
        pltpu.sync_copy(stage, out.at[c, pl.ds(r0, 128)])


def _sc_aggregate(table, src3, dst3, nb):
    body = functools.partial(_agg_body, nb)
    k = pl.kernel(
        body,
        out_type=jax.ShapeDtypeStruct((_NC, _NP, 64), jnp.float32),
        mesh=_sc_mesh(),
        compiler_params=_sc_params(),
        scratch_types=[
            pltpu.VMEM((nb, _SB), jnp.int32),
            pltpu.VMEM((nb, _SB), jnp.int32),
            pltpu.VMEM((_SB, 64), jnp.float32),
            pltpu.VMEM((_SB, 64), jnp.float32),
            pltpu.VMEM((_SB, 64), jnp.float32),
            pltpu.VMEM((_SB, 64), jnp.float32),
            pltpu.VMEM((128, 64), jnp.float32),
            pltpu.VMEM_SHARED((_NP, 64), jnp.float32),
            pltpu.SemaphoreType.DMA,
        ],
    )
    return k(table, src3, dst3)


def _mm_body(x_ref, w_ref, o_ref):
    o_ref[...] = jnp.dot(x_ref[...], w_ref[...],
                         preferred_element_type=jnp.float32)


def _scale_body(degp_ref, h1_ref, dis_ref, hs1_ref):
    n = h1_ref.shape[0]
    degp = degp_ref[...]
    deg = degp[0, :n, 0] + degp[1, :n, 0] + 1.0
    dis = lax.rsqrt(deg)
    dis_ref[...] = dis
    hs1_ref[...] = h1_ref[...] * dis[:, None]


def _hidden_body(rawp_ref, h1_ref, dis_ref, b1_ref, h_ref, hs2_ref):
    n = h1_ref.shape[0]
    rawp = rawp_ref[...]
    raw = rawp[0, :n] + rawp[1, :n]
    dis = dis_ref[...]
    a1 = dis[:, None] * raw + (dis * dis)[:, None] * h1_ref[...] \
        + b1_ref[...][None, :]
    h = jnp.maximum(a1, 0.0)
    h_ref[...] = h
    hs2_ref[...] = h * dis[:, None]


def _head_body(rawp_ref, h_ref, dis_ref, wmu_ref, bmu_ref, wvar_ref,
               bvar_ref, eps_ref, zm_ref, zv_ref, z_ref):
    n = h_ref.shape[0]
    rawp = rawp_ref[...]
    raw = rawp[0, :n] + rawp[1, :n]
    dis = dis_ref[...]
    a2 = dis[:, None] * raw + (dis * dis)[:, None] * h_ref[...]
    zm = jnp.dot(a2, wmu_ref[...], preferred_element_type=jnp.float32) \
        + bmu_ref[...][None, :]
    pv = jnp.dot(a2, wvar_ref[...], preferred_element_type=jnp.float32) \
        + bvar_ref[...][None, :]
    zv = jnp.maximum(pv, 0.0) + jnp.log(1.0 + jnp.exp(-jnp.abs(pv)))
    zm_ref[...] = zm
    zv_ref[...] = zv
    z_ref[...] = zm + zv * eps_ref[...]


def kernel(x, edge_index, W1, b1, Wmu, bmu, Wvar, bvar):
    n, d = x.shape
    e = edge_index.shape[1]
    h = W1.shape[1]
    z = Wmu.shape[1]
    nb = e // (_NW * _SB)
    src3 = edge_index[0].reshape(_NW, nb, _SB)
    dst3 = edge_index[1].reshape(_NW, nb, _SB)

    f32 = jnp.float32
    h1 = pl.pallas_call(
        _mm_body, out_shape=jax.ShapeDtypeStruct((n, h), f32))(x, W1)

    degp = _sc_degree(dst3, nb)

    dis, hs1 = pl.pallas_call(
        _scale_body,
        out_shape=(jax.ShapeDtypeStruct((n,), f32),
                   jax.ShapeDtypeStruct((n, h), f32)))(degp, h1)

    raw1p = _sc_aggregate(hs1, src3, dst3, nb)

    hh, hs2 = pl.pallas_call(
        _hidden_body,
        out_shape=(jax.ShapeDtypeStruct((n, h), f32),
                   jax.ShapeDtypeStruct((n, h), f32)))(raw1p, h1, dis, b1)

    raw2p = _sc_aggregate(hs2, src3, dst3, nb)

    with jax.ensure_compile_time_eval():
        eps = jax.random.normal(jax.random.key(42), (n, z), f32)
    zm, zv, zz = pl.pallas_call(
        _head_body,
        out_shape=(jax.ShapeDtypeStruct((n, z), f32),
                   jax.ShapeDtypeStruct((n, z), f32),
                   jax.ShapeDtypeStruct((n, z), f32)))(
        raw2p, hh, dis, Wmu, bmu, Wvar, bvar, eps)
    return (zm, zv, zz)


# speedup vs baseline: 2.9691x; 1.0608x over previous
import functools

import jax
import jax.numpy as jnp
from jax import lax
from jax.experimental import pallas as pl
from jax.experimental.pallas import tpu as pltpu
from jax.experimental.pallas import tpu_sc as plsc

_NC = 2
_NS = 16
_NW = _NC * _NS
_SB = 80
_L = 16
_NP = 10240


def _sc_mesh():
    return plsc.VectorSubcoreMesh(core_axis_name="c", subcore_axis_name="s")


def _sc_params():
    return pltpu.CompilerParams(use_tc_tiling_on_sc=False,
                                needs_layout_passes=False)


def _deg_body(nb, dst3, out, dst2d, ones_v, stage, cbuf, acc):
    c = lax.axis_index("c")
    s = lax.axis_index("s")
    w = c * _NS + s
    stripe = _NP // _NS
    nchunk = stripe // 128

    zeros = jnp.zeros((_L,), jnp.float32)
    ones = jnp.ones((_L,), jnp.float32)
    for i in range(128):
        stage[i, :] = zeros
    for i in range(_SB):
        ones_v[i, :] = ones
    for k in range(nchunk):
        pltpu.sync_copy(stage, acc.at[pl.ds(s * stripe + k * 128, 128)])
    plsc.subcore_barrier()

    pltpu.sync_copy(dst3.at[w], dst2d)

    def body(j, carry):
        pltpu.sync_copy(ones_v, acc.at[dst2d.at[j]], add=True)
        return carry

    lax.fori_loop(0, nb, body, 0)
    plsc.subcore_barrier()

    iota = lax.iota(jnp.int32, _L)
    zidx = jnp.zeros((_L,), jnp.int32)
    for k in range(nchunk):
        r0 = s * stripe + k * 128
        pltpu.sync_copy(acc.at[pl.ds(r0, 128)], stage)
        for g in range(128 // _L):
            v = plsc.load_gather(stage, [g * _L + iota, zidx])
            cbuf[pl.ds(k * 128 + g * _L, _L)] = v
    pltpu.sync_copy(cbuf, out.at[c, pl.ds(s * stripe, stripe)])


def _sc_degree(dst3, nb):
    body = functools.partial(_deg_body, nb)
    k = pl.kernel(
        body,
        out_type=jax.ShapeDtypeStruct((_NC, _NP), jnp.float32),
        mesh=_sc_mesh(),
        compiler_params=_sc_params(),
        scratch_types=[
            pltpu.VMEM((nb, _SB), jnp.int32),
            pltpu.VMEM((_SB, _L), jnp.float32),
            pltpu.VMEM((128, _L), jnp.float32),
            pltpu.VMEM((_NP // _NS,), jnp.float32),
            pltpu.VMEM_SHARED((_NP, _L), jnp.float32),
        ],
    )
    return k(dst3)


def _agg_body(nb, table, src3, dst3, out, src2d, dst2d, rows0, rows1, rows2,
              rows3, stage, acc, sem):
    c = lax.axis_index("c")
    s = lax.axis_index("s")
    w = c * _NS + s
    stripe = _NP // _NS
    nchunk = stripe // 128
    bufs = [rows0, rows1, rows2, rows3]
    depth = 3

    zeros = jnp.zeros((_L,), jnp.float32)
    for i in range(128):
        for cc in range(4):
            stage[i, pl.ds(cc * _L, _L)] = zeros
    for k in range(nchunk):
        pltpu.sync_copy(stage, acc.at[pl.ds(s * stripe + k * 128, 128)])
    plsc.subcore_barrier()

    pltpu.sync_copy(src3.at[w], src2d)
    pltpu.sync_copy(dst3.at[w], dst2d)

    for j in range(depth):
        pltpu.async_copy(table.at[src2d.at[j]], bufs[j], sem)

    nb4 = nb // 4
    rem = nb - 4 * nb4

    def body(i, carry):
        j = 4 * i
        for k in range(4):
            b = bufs[k]
            pltpu.make_async_copy(table.at[src2d.at[j]], b, sem).wait()

            @pl.when(j + k + depth < nb)
            def _():
                pltpu.async_copy(
                    table.at[src2d.at[j + k + depth]], bufs[(k + depth) % 4],
                    sem)

            pltpu.sync_copy(b, acc.at[dst2d.at[j + k]], add=True)
        return carry

    lax.fori_loop(0, nb4, body, 0)
    for t in range(rem):
        j = 4 * nb4 + t
        b = bufs[j % 4]
        pltpu.make_async_copy(table.at[src2d.at[0]], b, sem).wait()
        pltpu.sync_copy(b, acc.at[dst2d.at[j]], add=True)
    plsc.subcore_barrier()

    for k in range(nchunk):
        r0 = s * stripe + k * 128
        pltpu.sync_copy(acc.at[pl.ds(r0, 128)], stage)
        pltpu.sync_copy(stage, out.at[c, pl.ds(r0, 128)])


def _sc_aggregate(table, src3, dst3, nb):
    body = functools.partial(_agg_body, nb)
    k = pl.kernel(
        body,
        out_type=jax.ShapeDtypeStruct((_NC, _NP, 64), jnp.float32),
        mesh=_sc_mesh(),
        compiler_params=_sc_params(),
        scratch_types=[
            pltpu.VMEM((nb, _SB), jnp.int32),
            pltpu.VMEM((nb, _SB), jnp.int32),
            pltpu.VMEM((_SB, 64), jnp.float32),
            pltpu.VMEM((_SB, 64), jnp.float32),
            pltpu.VMEM((_SB, 64), jnp.float32),
            pltpu.VMEM((_SB, 64), jnp.float32),
            pltpu.VMEM((128, 64), jnp.float32),
            pltpu.VMEM_SHARED((_NP, 64), jnp.float32),
            pltpu.SemaphoreType.DMA,
        ],
    )
    return k(table, src3, dst3)


_BR = 2048


def _mm_body(x_ref, w_ref, o_ref):
    o_ref[...] = jnp.dot(x_ref[...], w_ref[...],
                         preferred_element_type=jnp.float32)


def _scale_body(degp_ref, h1_ref, dis_ref, hs1_ref):
    degp = degp_ref[...]
    deg = degp[0] + degp[1] + 1.0
    dis = lax.rsqrt(deg)
    dis_ref[...] = dis
    hs1_ref[...] = h1_ref[...] * dis[:, None]


def _hidden_body(rawp_ref, h1_ref, dis_ref, b1_ref, h_ref, hs2_ref):
    rawp = rawp_ref[...]
    raw = rawp[0] + rawp[1]
    dis = dis_ref[...]
    a1 = dis[:, None] * raw + (dis * dis)[:, None] * h1_ref[...] \
        + b1_ref[...][None, :]
    h = jnp.maximum(a1, 0.0)
    h_ref[...] = h
    hs2_ref[...] = h * dis[:, None]


def _head_body(rawp_ref, h_ref, dis_ref, wmu_ref, bmu_ref, wvar_ref,
               bvar_ref, eps_ref, zm_ref, zv_ref, z_ref):
    rawp = rawp_ref[...]
    raw = rawp[0] + rawp[1]
    dis = dis_ref[...]
    a2 = dis[:, None] * raw + (dis * dis)[:, None] * h_ref[...]
    zm = jnp.dot(a2, wmu_ref[...], preferred_element_type=jnp.float32) \
        + bmu_ref[...][None, :]
    pv = jnp.dot(a2, wvar_ref[...], preferred_element_type=jnp.float32) \
        + bvar_ref[...][None, :]
    zv = jnp.maximum(pv, 0.0) + jnp.log(1.0 + jnp.exp(-jnp.abs(pv)))
    zm_ref[...] = zm
    zv_ref[...] = zv
    z_ref[...] = zm + zv * eps_ref[...]


def _row_spec(width=None):
    if width is None:
        return pl.BlockSpec((_BR,), lambda i: (i,))
    return pl.BlockSpec((_BR, width), lambda i: (i, 0))


def _full_spec(shape):
    nd = len(shape)
    return pl.BlockSpec(shape, lambda i: (0,) * nd)


def _part_spec(width):
    return pl.BlockSpec((2, _BR, width), lambda i: (0, i, 0))


def kernel(x, edge_index, W1, b1, Wmu, bmu, Wvar, bvar):
    n, d = x.shape
    e = edge_index.shape[1]
    h = W1.shape[1]
    z = Wmu.shape[1]
    nb = e // (_NW * _SB)
    src3 = edge_index[0].reshape(_NW, nb, _SB)
    dst3 = edge_index[1].reshape(_NW, nb, _SB)

    f32 = jnp.float32
    grid = ((n + _BR - 1) // _BR,)
    h1 = pl.pallas_call(
        _mm_body, grid=grid,
        in_specs=[_row_spec(d), _full_spec((d, h))],
        out_specs=_row_spec(h),
        out_shape=jax.ShapeDtypeStruct((n, h), f32))(x, W1)

    degp = _sc_degree(dst3, nb)

    dis, hs1 = pl.pallas_call(
        _scale_body, grid=grid,
        in_specs=[pl.BlockSpec((2, _BR), lambda i: (0, i)), _row_spec(h)],
        out_specs=(_row_spec(), _row_spec(h)),
        out_shape=(jax.ShapeDtypeStruct((n,), f32),
                   jax.ShapeDtypeStruct((n, h), f32)))(degp, h1)

    raw1p = _sc_aggregate(hs1, src3, dst3, nb)

    hh, hs2 = pl.pallas_call(
        _hidden_body, grid=grid,
        in_specs=[_part_spec(h), _row_spec(h), _row_spec(),
                  _full_spec((h,))],
        out_specs=(_row_spec(h), _row_spec(h)),
        out_shape=(jax.ShapeDtypeStruct((n, h), f32),
                   jax.ShapeDtypeStruct((n, h), f32)))(raw1p, h1, dis, b1)

    raw2p = _sc_aggregate(hs2, src3, dst3, nb)

    with jax.ensure_compile_time_eval():
        eps = jax.random.normal(jax.random.key(42), (n, z), f32)
    zm, zv, zz = pl.pallas_call(
        _head_body, grid=grid,
        in_specs=[_part_spec(h), _row_spec(h), _row_spec(),
                  _full_spec((h, z)), _full_spec((z,)),
                  _full_spec((h, z)), _full_spec((z,)), _row_spec(z)],
        out_specs=(_row_spec(z), _row_spec(z), _row_spec(z)),
        out_shape=(jax.ShapeDtypeStruct((n, z), f32),
                   jax.ShapeDtypeStruct((n, z), f32),
                   jax.ShapeDtypeStruct((n, z), f32)))(
        raw2p, hh, dis, Wmu, bmu, Wvar, bvar, eps)
    return (zm, zv, zz)
